# baseline (speedup 1.0000x reference)
import functools

import jax
import jax.numpy as jnp
from jax import lax
from jax.experimental import pallas as pl
from jax.experimental.pallas import tpu as pltpu
from jax.experimental.pallas import tpu_sc as plsc

N = 10000
E = 160000
D = 256
DH = 128
NC = 2
NS = 16
L = 16

K = 128
NBUF = 4
NPASS = 2
HN = N // NPASS
AROWS = 5120
ROWS_PT = AROWS // NS

BCAP = 5632
BCH = BCAP // K
BGRP = BCH // NBUF
BBUF = BCAP + L

DPT = E // NS
NPAD = 128
DEG_RPT = NPAD // NS

RB = 1000
NB = N // RB
BPP = HN // RB

@functools.cache
def _mesh():
    return plsc.VectorSubcoreMesh(
        core_axis_name="c", subcore_axis_name="s",
        num_cores=NC, num_subcores=NS)


def _deg_body(ei_ref, out_ref, ibuf, acc, zbuf, iden, shared):
    c = lax.axis_index("c")
    s = lax.axis_index("s")

    z16 = jnp.zeros((L,), jnp.float32)

    @pl.loop(0, DEG_RPT)
    def _(i):
        for v in range(128 // L):
            zbuf[i, pl.ds(v * L, L)] = z16

    pltpu.sync_copy(zbuf, shared.at[pl.ds(s * DEG_RPT, DEG_RPT)])

    @pl.loop(0, NPAD)
    def _(i):
        for v in range(128 // L):
            acc[i, pl.ds(v * L, L)] = z16

    pltpu.sync_copy(ei_ref.at[pl.ds(c * E + s * DPT, DPT)], ibuf)

    ones = jnp.ones((L,), jnp.float32)

    @pl.loop(0, DPT // L)
    def _(j):
        idx = ibuf[pl.ds(j * L, L)]
        hi = lax.shift_right_logical(idx, 7)
        lo = lax.bitwise_and(idx, 127)
        plsc.addupdate_scatter(acc, [hi, lo], ones)

    iota = lax.iota(jnp.int32, L)
    for v in range(128 // L):
        iden[0, pl.ds(v * L, L)] = iota + v * L

    plsc.subcore_barrier()
    pltpu.sync_copy(acc, shared.at[iden.at[0]], add=True)
    plsc.subcore_barrier()

    pltpu.sync_copy(shared.at[pl.ds(s * DEG_RPT, DEG_RPT)],
                    out_ref.at[c, pl.ds(s * DEG_RPT, DEG_RPT)])


@functools.cache
def _deg_call():
    return pl.kernel(
        _deg_body,
        out_type=jax.ShapeDtypeStruct((NC, NPAD, 128), jnp.float32),
        mesh=_mesh(),
        scratch_types=[
            pltpu.VMEM((DPT,), jnp.int32),
            pltpu.VMEM((NPAD, 128), jnp.float32),
            pltpu.VMEM((DEG_RPT, 128), jnp.float32),
            pltpu.VMEM((1, 128), jnp.int32),
            pltpu.VMEM_SHARED((NPAD, 128), jnp.float32),
        ],
        compiler_params=pltpu.CompilerParams(needs_layout_passes=False),
    )


def _bin_body(ei_ref, bsrc_ref, bdst_ref, sstage, dstage, sbin, dbin):
    c = lax.axis_index("c")
    s = lax.axis_index("s")

    pltpu.sync_copy(ei_ref.at[pl.ds(s * DPT, DPT)], sstage)
    pltpu.sync_copy(ei_ref.at[pl.ds(E + s * DPT, DPT)], dstage)

    iota = lax.iota(jnp.int32, L)

    @pl.loop(0, BBUF // L)
    def _(j):
        sbin[pl.ds(j * L, L)] = lax.rem(iota + j * L, N)
        dbin[pl.ds(j * L, L)] = HN + lax.rem(iota * 8 + j, 120)

    base = c * HN

    @pl.loop(0, DPT // L, init_carry=jnp.int32(0))
    def _(j, off):
        sv = sstage[pl.ds(j * L, L)]
        dl = dstage[pl.ds(j * L, L)] - base
        m = jnp.logical_and(dl >= 0, dl < HN)
        plsc.store_compressed(sbin.at[pl.ds(off, L)], sv, mask=m)
        plsc.store_compressed(dbin.at[pl.ds(off, L)], dl, mask=m)
        cnt = plsc.all_reduce_population_count(m)
        return off + cnt[0]

    pltpu.sync_copy(sbin.at[pl.ds(0, BCAP)], bsrc_ref.at[c, s])
    pltpu.sync_copy(dbin.at[pl.ds(0, BCAP)], bdst_ref.at[c, s])


@functools.cache
def _bin_call():
    return pl.kernel(
        _bin_body,
        out_type=(
            jax.ShapeDtypeStruct((NPASS, NS, BCAP), jnp.int32),
            jax.ShapeDtypeStruct((NPASS, NS, BCAP), jnp.int32),
        ),
        mesh=_mesh(),
        scratch_types=[
            pltpu.VMEM((DPT,), jnp.int32),
            pltpu.VMEM((DPT,), jnp.int32),
            pltpu.VMEM((BBUF,), jnp.int32),
            pltpu.VMEM((BBUF,), jnp.int32),
        ],
        compiler_params=pltpu.CompilerParams(needs_layout_passes=False),
    )


def _agg_body(m_ref, bsrc_ref, bdst_ref, zeros_ref, out_ref,
              sbuf, vbuf, rows, acc, gsem, ssem):
    c = lax.axis_index("c")
    s = lax.axis_index("s")

    cN = c * N

    for p in range(NPASS):
        pltpu.sync_copy(bsrc_ref.at[p, s], sbuf)
        pltpu.sync_copy(bdst_ref.at[p, s], vbuf)

        @pl.loop(0, BCH)
        def _(i):
            for v in range(K // L):
                sl = pl.ds(v * L, L)
                sbuf[i, sl] = sbuf[i, sl] + cN

        for b in range(NBUF):
            pltpu.async_copy(m_ref.at[sbuf.at[b]], rows.at[b], gsem.at[b])

        pltpu.sync_copy(zeros_ref, acc.at[pl.ds(s * ROWS_PT, ROWS_PT)])

        plsc.subcore_barrier()

        @pl.loop(0, BGRP)
        def _(g):
            for b in range(NBUF):
                i = g * NBUF + b
                pltpu.make_async_copy(
                    m_ref.at[sbuf.at[i]], rows.at[b], gsem.at[b]).wait()
                pltpu.async_copy(rows.at[b], acc.at[vbuf.at[i]], ssem.at[b],
                                 add=True)
            for b in range(NBUF):
                i = g * NBUF + b
                pltpu.make_async_copy(
                    rows.at[b], acc.at[vbuf.at[i]], ssem.at[b]).wait()

                @pl.when(g < BGRP - 1)
                def _():
                    inext = (g + 1) * NBUF + b
                    pltpu.async_copy(
                        m_ref.at[sbuf.at[inext]], rows.at[b], gsem.at[b])

        plsc.subcore_barrier()
        pltpu.sync_copy(acc.at[pl.ds(s * ROWS_PT, ROWS_PT)],
                        out_ref.at[c, p, pl.ds(s * ROWS_PT, ROWS_PT)])
        if p + 1 < NPASS:
            plsc.subcore_barrier()


@functools.cache
def _agg_call():
    return pl.kernel(
        _agg_body,
        out_type=jax.ShapeDtypeStruct((NC, NPASS, AROWS, DH), jnp.float32),
        mesh=_mesh(),
        scratch_types=[
            pltpu.VMEM((BCH, K), jnp.int32),
            pltpu.VMEM((BCH, K), jnp.int32),
            pltpu.VMEM((NBUF, K, DH), jnp.float32),
            pltpu.VMEM_SHARED((AROWS, DH), jnp.float32),
            pltpu.SemaphoreType.DMA((NBUF,)),
            pltpu.SemaphoreType.DMA((NBUF,)),
        ],
        compiler_params=pltpu.CompilerParams(needs_layout_passes=False),
    )


def _norm(dcol):
    return jnp.where(dcol > 0, lax.rsqrt(dcol), 0.0)


def _dot_bf16(x, w):
    return jnp.dot(x.astype(jnp.bfloat16), w.astype(jnp.bfloat16),
                   preferred_element_type=jnp.float32)


def _mm1_body(x_ref, w_ref, nod_ref, o_ref):
    no = _norm(nod_ref[0])
    o_ref[...] = _dot_bf16(x_ref[...], w_ref[...]) * no


def _mm1(features, w1, outdeg_col):
    return pl.pallas_call(
        _mm1_body,
        grid=(NB, NC),
        in_specs=[
            pl.BlockSpec((RB, D), lambda i, c: (i, 0)),
            pl.BlockSpec((D, DH), lambda i, c: (0, c)),
            pl.BlockSpec((1, RB, 1), lambda i, c: (i, 0, 0)),
        ],
        out_specs=pl.BlockSpec((RB, DH), lambda i, c: (c * NB + i, 0)),
        out_shape=jax.ShapeDtypeStruct((2 * N, DH), jnp.float32),
    )(features, w1, outdeg_col)


def _half_spec(ci):
    return pl.BlockSpec((1, 1, RB, DH),
                        lambda *g: (ci, g[0] // BPP, g[0] % BPP, 0))


def _mm2_body(a0_ref, a1_ref, nid_ref, b1_ref, w_ref, nod_ref, o_ref):
    ni = _norm(nid_ref[0])
    no = _norm(nod_ref[0])
    h = jnp.concatenate([a0_ref[0, 0], a1_ref[0, 0]], axis=1)
    h = jnp.maximum(h * ni + b1_ref[...], 0.0)
    o_ref[...] = _dot_bf16(h, w_ref[...]) * no


def _mm2(agg1, indeg_col, b1_2d, w2, outdeg_col):
    return pl.pallas_call(
        _mm2_body,
        grid=(NB, NC),
        in_specs=[
            _half_spec(0),
            _half_spec(1),
            pl.BlockSpec((1, RB, 1), lambda i, c: (i, 0, 0)),
            pl.BlockSpec((1, D), lambda i, c: (0, 0)),
            pl.BlockSpec((D, DH), lambda i, c: (0, c)),
            pl.BlockSpec((1, RB, 1), lambda i, c: (i, 0, 0)),
        ],
        out_specs=pl.BlockSpec((RB, DH), lambda i, c: (c * NB + i, 0)),
        out_shape=jax.ShapeDtypeStruct((2 * N, DH), jnp.float32),
    )(agg1, agg1, indeg_col, b1_2d, w2, outdeg_col)


def _epi_body(a0_ref, a1_ref, nid_ref, b2_ref, o_ref):
    ni = _norm(nid_ref[0])
    agg = jnp.concatenate([a0_ref[0, 0], a1_ref[0, 0]], axis=1)
    o_ref[...] = agg * ni + b2_ref[...]


def _epi(agg2, indeg_col, b2_2d):
    return pl.pallas_call(
        _epi_body,
        grid=(NB,),
        in_specs=[
            _half_spec(0),
            _half_spec(1),
            pl.BlockSpec((1, RB, 1), lambda i: (i, 0, 0)),
            pl.BlockSpec((1, D), lambda i: (0, 0)),
        ],
        out_specs=pl.BlockSpec((RB, D), lambda i: (i, 0)),
        out_shape=jax.ShapeDtypeStruct((N, D), jnp.float32),
    )(agg2, agg2, indeg_col, b2_2d)


def kernel(features, edge_index, W1, b1, W2, b2):
    ei_flat = edge_index.reshape(-1)

    deg = _deg_call()(ei_flat)
    outdeg_col = deg[0].reshape(-1)[:N].reshape(NB, RB, 1)
    indeg_col = deg[1].reshape(-1)[:N].reshape(NB, RB, 1)

    bsrc, bdst = _bin_call()(ei_flat)
    bsrc = bsrc.reshape(NPASS, NS, BCH, K)
    bdst = bdst.reshape(NPASS, NS, BCH, K)

    zeros = jnp.zeros((ROWS_PT, DH), jnp.float32)
    m1 = _mm1(features, W1, outdeg_col)
    agg1 = _agg_call()(m1, bsrc, bdst, zeros)
    m2 = _mm2(agg1, indeg_col, b1.reshape(1, D), W2, outdeg_col)
    agg2 = _agg_call()(m2, bsrc, bdst, zeros)
    return _epi(agg2, indeg_col, b2.reshape(1, D))


# speedup vs baseline: 1.0265x; 1.0265x over previous
import functools

import jax
import jax.numpy as jnp
from jax import lax
from jax.experimental import pallas as pl
from jax.experimental.pallas import tpu as pltpu
from jax.experimental.pallas import tpu_sc as plsc

N = 10000
E = 160000
D = 256
DH = 128
NC = 2
NS = 16
L = 16

K = 128
NBUF = 4
NPASS = 2
HN = N // NPASS
AROWS = 5120
ROWS_PT = AROWS // NS

BCAP = 10240
BCH = BCAP // K
GSZ = NBUF * K
BBUF = BCAP + L

DPT = E // NS
NPAD = 128
DEG_RPT = NPAD // NS

RB = 1000
NB = N // RB
BPP = HN // RB

@functools.cache
def _mesh():
    return plsc.VectorSubcoreMesh(
        core_axis_name="c", subcore_axis_name="s",
        num_cores=NC, num_subcores=NS)


def _deg_body(ei_ref, out_ref, ibuf, acc, zbuf, iden, shared):
    c = lax.axis_index("c")
    s = lax.axis_index("s")

    z16 = jnp.zeros((L,), jnp.float32)

    @pl.loop(0, DEG_RPT)
    def _(i):
        for v in range(128 // L):
            zbuf[i, pl.ds(v * L, L)] = z16

    pltpu.sync_copy(zbuf, shared.at[pl.ds(s * DEG_RPT, DEG_RPT)])

    @pl.loop(0, NPAD)
    def _(i):
        for v in range(128 // L):
            acc[i, pl.ds(v * L, L)] = z16

    pltpu.sync_copy(ei_ref.at[pl.ds(c * E + s * DPT, DPT)], ibuf)

    ones = jnp.ones((L,), jnp.float32)

    @pl.loop(0, DPT // L)
    def _(j):
        idx = ibuf[pl.ds(j * L, L)]
        hi = lax.shift_right_logical(idx, 7)
        lo = lax.bitwise_and(idx, 127)
        plsc.addupdate_scatter(acc, [hi, lo], ones)

    iota = lax.iota(jnp.int32, L)
    for v in range(128 // L):
        iden[0, pl.ds(v * L, L)] = iota + v * L

    plsc.subcore_barrier()
    pltpu.sync_copy(acc, shared.at[iden.at[0]], add=True)
    plsc.subcore_barrier()

    pltpu.sync_copy(shared.at[pl.ds(s * DEG_RPT, DEG_RPT)],
                    out_ref.at[c, pl.ds(s * DEG_RPT, DEG_RPT)])


@functools.cache
def _deg_call():
    return pl.kernel(
        _deg_body,
        out_type=jax.ShapeDtypeStruct((NC, NPAD, 128), jnp.float32),
        mesh=_mesh(),
        scratch_types=[
            pltpu.VMEM((DPT,), jnp.int32),
            pltpu.VMEM((NPAD, 128), jnp.float32),
            pltpu.VMEM((DEG_RPT, 128), jnp.float32),
            pltpu.VMEM((1, 128), jnp.int32),
            pltpu.VMEM_SHARED((NPAD, 128), jnp.float32),
        ],
        compiler_params=pltpu.CompilerParams(needs_layout_passes=False),
    )


def _bin_body(ei_ref, bsrc_ref, bdst_ref, cnt_ref,
              sstage, dstage, sbin, dbin, cbuf):
    c = lax.axis_index("c")
    s = lax.axis_index("s")

    pltpu.sync_copy(ei_ref.at[pl.ds(s * DPT, DPT)], sstage)
    pltpu.sync_copy(ei_ref.at[pl.ds(E + s * DPT, DPT)], dstage)

    iota = lax.iota(jnp.int32, L)

    @pl.loop(0, BBUF // L)
    def _(j):
        sbin[pl.ds(j * L, L)] = lax.rem(iota + j * L, N)
        dbin[pl.ds(j * L, L)] = HN + lax.rem(iota * 8 + j, 120)

    base = c * HN

    @pl.loop(0, DPT // L, init_carry=jnp.int32(0))
    def off_final(j, off):
        sv = sstage[pl.ds(j * L, L)]
        dl = dstage[pl.ds(j * L, L)] - base
        m = jnp.logical_and(dl >= 0, dl < HN)
        plsc.store_compressed(sbin.at[pl.ds(off, L)], sv, mask=m)
        plsc.store_compressed(dbin.at[pl.ds(off, L)], dl, mask=m)
        cnt = plsc.all_reduce_population_count(m)
        return off + cnt[0]

    cbuf[pl.ds(0, L)] = jnp.zeros((L,), jnp.int32) + off_final
    pltpu.sync_copy(cbuf, cnt_ref.at[c, s])
    pltpu.sync_copy(sbin.at[pl.ds(0, BCAP)], bsrc_ref.at[c, s])
    pltpu.sync_copy(dbin.at[pl.ds(0, BCAP)], bdst_ref.at[c, s])


@functools.cache
def _bin_call():
    return pl.kernel(
        _bin_body,
        out_type=(
            jax.ShapeDtypeStruct((NPASS, NS, BCAP), jnp.int32),
            jax.ShapeDtypeStruct((NPASS, NS, BCAP), jnp.int32),
            jax.ShapeDtypeStruct((NPASS, NS, L), jnp.int32),
        ),
        mesh=_mesh(),
        scratch_types=[
            pltpu.VMEM((DPT,), jnp.int32),
            pltpu.VMEM((DPT,), jnp.int32),
            pltpu.VMEM((BBUF,), jnp.int32),
            pltpu.VMEM((BBUF,), jnp.int32),
            pltpu.VMEM((L,), jnp.int32),
        ],
        compiler_params=pltpu.CompilerParams(needs_layout_passes=False),
    )


def _agg_body(m_ref, bsrc_ref, bdst_ref, cnt_ref, zeros_ref, out_ref,
              sbuf, vbuf, rows, cbuf, acc, gsem, ssem):
    c = lax.axis_index("c")
    s = lax.axis_index("s")

    cN = c * N

    for p in range(NPASS):
        pltpu.sync_copy(bsrc_ref.at[p, s], sbuf)
        pltpu.sync_copy(bdst_ref.at[p, s], vbuf)
        pltpu.sync_copy(cnt_ref.at[p, s], cbuf)

        cnt = cbuf[pl.ds(0, L)][0]
        ng = jnp.maximum(lax.shift_right_logical(cnt + (GSZ - 1), 9), 1)

        @pl.loop(0, BCH)
        def _(i):
            for v in range(K // L):
                sl = pl.ds(v * L, L)
                sbuf[i, sl] = sbuf[i, sl] + cN

        for b in range(NBUF):
            pltpu.async_copy(m_ref.at[sbuf.at[b]], rows.at[b], gsem.at[b])

        pltpu.sync_copy(zeros_ref, acc.at[pl.ds(s * ROWS_PT, ROWS_PT)])

        plsc.subcore_barrier()

        @pl.loop(0, ng)
        def _(g):
            for b in range(NBUF):
                i = g * NBUF + b
                pltpu.make_async_copy(
                    m_ref.at[sbuf.at[i]], rows.at[b], gsem.at[b]).wait()
                pltpu.async_copy(rows.at[b], acc.at[vbuf.at[i]], ssem.at[b],
                                 add=True)
            for b in range(NBUF):
                i = g * NBUF + b
                pltpu.make_async_copy(
                    rows.at[b], acc.at[vbuf.at[i]], ssem.at[b]).wait()

                @pl.when(g < ng - 1)
                def _():
                    inext = (g + 1) * NBUF + b
                    pltpu.async_copy(
                        m_ref.at[sbuf.at[inext]], rows.at[b], gsem.at[b])

        plsc.subcore_barrier()
        pltpu.sync_copy(acc.at[pl.ds(s * ROWS_PT, ROWS_PT)],
                        out_ref.at[c, p, pl.ds(s * ROWS_PT, ROWS_PT)])
        if p + 1 < NPASS:
            plsc.subcore_barrier()


@functools.cache
def _agg_call():
    return pl.kernel(
        _agg_body,
        out_type=jax.ShapeDtypeStruct((NC, NPASS, AROWS, DH), jnp.float32),
        mesh=_mesh(),
        scratch_types=[
            pltpu.VMEM((BCH, K), jnp.int32),
            pltpu.VMEM((BCH, K), jnp.int32),
            pltpu.VMEM((NBUF, K, DH), jnp.float32),
            pltpu.VMEM((L,), jnp.int32),
            pltpu.VMEM_SHARED((AROWS, DH), jnp.float32),
            pltpu.SemaphoreType.DMA((NBUF,)),
            pltpu.SemaphoreType.DMA((NBUF,)),
        ],
        compiler_params=pltpu.CompilerParams(needs_layout_passes=False),
    )


def _norm(dcol):
    return jnp.where(dcol > 0, lax.rsqrt(dcol), 0.0)


def _mm1_body(x_ref, w_ref, nod_ref, o_ref):
    no = _norm(nod_ref[0])
    hw = jnp.dot(x_ref[...], w_ref[...], preferred_element_type=jnp.float32)
    o_ref[...] = hw * no


def _mm1(features, w1, outdeg_col):
    return pl.pallas_call(
        _mm1_body,
        grid=(NB, NC),
        in_specs=[
            pl.BlockSpec((RB, D), lambda i, c: (i, 0)),
            pl.BlockSpec((D, DH), lambda i, c: (0, c)),
            pl.BlockSpec((1, RB, 1), lambda i, c: (i, 0, 0)),
        ],
        out_specs=pl.BlockSpec((RB, DH), lambda i, c: (c * NB + i, 0)),
        out_shape=jax.ShapeDtypeStruct((2 * N, DH), jnp.float32),
    )(features, w1, outdeg_col)


def _half_spec(ci):
    return pl.BlockSpec((1, 1, RB, DH),
                        lambda *g: (ci, g[0] // BPP, g[0] % BPP, 0))


def _mm2_body(a0_ref, a1_ref, nid_ref, b1_ref, w_ref, nod_ref, o_ref):
    ni = _norm(nid_ref[0])
    no = _norm(nod_ref[0])
    h = jnp.concatenate([a0_ref[0, 0], a1_ref[0, 0]], axis=1)
    h = jnp.maximum(h * ni + b1_ref[...], 0.0)
    hw = jnp.dot(h, w_ref[...], preferred_element_type=jnp.float32)
    o_ref[...] = hw * no


def _mm2(agg1, indeg_col, b1_2d, w2, outdeg_col):
    return pl.pallas_call(
        _mm2_body,
        grid=(NB, NC),
        in_specs=[
            _half_spec(0),
            _half_spec(1),
            pl.BlockSpec((1, RB, 1), lambda i, c: (i, 0, 0)),
            pl.BlockSpec((1, D), lambda i, c: (0, 0)),
            pl.BlockSpec((D, DH), lambda i, c: (0, c)),
            pl.BlockSpec((1, RB, 1), lambda i, c: (i, 0, 0)),
        ],
        out_specs=pl.BlockSpec((RB, DH), lambda i, c: (c * NB + i, 0)),
        out_shape=jax.ShapeDtypeStruct((2 * N, DH), jnp.float32),
    )(agg1, agg1, indeg_col, b1_2d, w2, outdeg_col)


def _epi_body(a0_ref, a1_ref, nid_ref, b2_ref, o_ref):
    ni = _norm(nid_ref[0])
    agg = jnp.concatenate([a0_ref[0, 0], a1_ref[0, 0]], axis=1)
    o_ref[...] = agg * ni + b2_ref[...]


def _epi(agg2, indeg_col, b2_2d):
    return pl.pallas_call(
        _epi_body,
        grid=(NB,),
        in_specs=[
            _half_spec(0),
            _half_spec(1),
            pl.BlockSpec((1, RB, 1), lambda i: (i, 0, 0)),
            pl.BlockSpec((1, D), lambda i: (0, 0)),
        ],
        out_specs=pl.BlockSpec((RB, D), lambda i: (i, 0)),
        out_shape=jax.ShapeDtypeStruct((N, D), jnp.float32),
    )(agg2, agg2, indeg_col, b2_2d)


def kernel(features, edge_index, W1, b1, W2, b2):
    ei_flat = edge_index.reshape(-1)

    deg = _deg_call()(ei_flat)
    outdeg_col = deg[0].reshape(-1)[:N].reshape(NB, RB, 1)
    indeg_col = deg[1].reshape(-1)[:N].reshape(NB, RB, 1)

    bsrc, bdst, bcnt = _bin_call()(ei_flat)
    bsrc = bsrc.reshape(NPASS, NS, BCH, K)
    bdst = bdst.reshape(NPASS, NS, BCH, K)

    zeros = jnp.zeros((ROWS_PT, DH), jnp.float32)
    m1 = _mm1(features, W1, outdeg_col)
    agg1 = _agg_call()(m1, bsrc, bdst, bcnt, zeros)
    m2 = _mm2(agg1, indeg_col, b1.reshape(1, D), W2, outdeg_col)
    agg2 = _agg_call()(m2, bsrc, bdst, bcnt, zeros)
    return _epi(agg2, indeg_col, b2.reshape(1, D))


# speedup vs baseline: 1.0334x; 1.0067x over previous
import functools

import jax
import jax.numpy as jnp
from jax import lax
from jax.experimental import pallas as pl
from jax.experimental.pallas import tpu as pltpu
from jax.experimental.pallas import tpu_sc as plsc

N = 10000
E = 160000
D = 256
DH = 128
NC = 2
NS = 16
L = 16

K = 128
NBUF = 4
NPASS = 2
HN = N // NPASS
AROWS = 5120
ROWS_PT = AROWS // NS

BCAP = 10240
BCH = BCAP // K
GSZ = NBUF * K
BBUF = BCAP + L

DPT = E // NS
NPAD = 128
DEG_RPT = NPAD // NS

RB = 1000
NB = N // RB
BPP = HN // RB

@functools.cache
def _mesh():
    return plsc.VectorSubcoreMesh(
        core_axis_name="c", subcore_axis_name="s",
        num_cores=NC, num_subcores=NS)


def _bin_body(ei_ref, bsrc_ref, bdst_ref, cnt_ref, deg_ref,
              sstage, dstage, sbin, dbin, cbuf, hacc, hzbuf, iden, hshared):
    c = lax.axis_index("c")
    s = lax.axis_index("s")

    pltpu.sync_copy(ei_ref.at[pl.ds(s * DPT, DPT)], sstage)
    pltpu.sync_copy(ei_ref.at[pl.ds(E + s * DPT, DPT)], dstage)

    z16 = jnp.zeros((L,), jnp.float32)

    @pl.loop(0, DEG_RPT)
    def _(i):
        for v in range(128 // L):
            hzbuf[i, pl.ds(v * L, L)] = z16

    pltpu.sync_copy(hzbuf, hshared.at[pl.ds(s * DEG_RPT, DEG_RPT)])

    @pl.loop(0, NPAD)
    def _(i):
        for v in range(128 // L):
            hacc[i, pl.ds(v * L, L)] = z16

    ones = jnp.ones((L,), jnp.float32)

    @pl.when(c == 0)
    def _():
        @pl.loop(0, DPT // L)
        def _(j):
            idx = sstage[pl.ds(j * L, L)]
            plsc.addupdate_scatter(
                hacc, [lax.shift_right_logical(idx, 7),
                       lax.bitwise_and(idx, 127)], ones)

    @pl.when(c == 1)
    def _():
        @pl.loop(0, DPT // L)
        def _(j):
            idx = dstage[pl.ds(j * L, L)]
            plsc.addupdate_scatter(
                hacc, [lax.shift_right_logical(idx, 7),
                       lax.bitwise_and(idx, 127)], ones)

    hiota = lax.iota(jnp.int32, L)
    for v in range(128 // L):
        iden[0, pl.ds(v * L, L)] = hiota + v * L

    plsc.subcore_barrier()
    pltpu.sync_copy(hacc, hshared.at[iden.at[0]], add=True)
    plsc.subcore_barrier()
    pltpu.sync_copy(hshared.at[pl.ds(s * DEG_RPT, DEG_RPT)],
                    deg_ref.at[c, pl.ds(s * DEG_RPT, DEG_RPT)])

    iota = lax.iota(jnp.int32, L)

    @pl.loop(0, BBUF // L)
    def _(j):
        sbin[pl.ds(j * L, L)] = lax.rem(iota + j * L, N)
        dbin[pl.ds(j * L, L)] = HN + lax.rem(iota * 8 + j, 120)

    base = c * HN

    @pl.loop(0, DPT // L, init_carry=jnp.int32(0))
    def off_final(j, off):
        sv = sstage[pl.ds(j * L, L)]
        dl = dstage[pl.ds(j * L, L)] - base
        m = jnp.logical_and(dl >= 0, dl < HN)
        plsc.store_compressed(sbin.at[pl.ds(off, L)], sv, mask=m)
        plsc.store_compressed(dbin.at[pl.ds(off, L)], dl, mask=m)
        cnt = plsc.all_reduce_population_count(m)
        return off + cnt[0]

    cbuf[pl.ds(0, L)] = jnp.zeros((L,), jnp.int32) + off_final
    pltpu.sync_copy(cbuf, cnt_ref.at[c, s])
    pltpu.sync_copy(sbin.at[pl.ds(0, BCAP)], bsrc_ref.at[c, s])
    pltpu.sync_copy(dbin.at[pl.ds(0, BCAP)], bdst_ref.at[c, s])


@functools.cache
def _bin_call():
    return pl.kernel(
        _bin_body,
        out_type=(
            jax.ShapeDtypeStruct((NPASS, NS, BCAP), jnp.int32),
            jax.ShapeDtypeStruct((NPASS, NS, BCAP), jnp.int32),
            jax.ShapeDtypeStruct((NPASS, NS, L), jnp.int32),
            jax.ShapeDtypeStruct((NC, NPAD, 128), jnp.float32),
        ),
        mesh=_mesh(),
        scratch_types=[
            pltpu.VMEM((DPT,), jnp.int32),
            pltpu.VMEM((DPT,), jnp.int32),
            pltpu.VMEM((BBUF,), jnp.int32),
            pltpu.VMEM((BBUF,), jnp.int32),
            pltpu.VMEM((L,), jnp.int32),
            pltpu.VMEM((NPAD, 128), jnp.float32),
            pltpu.VMEM((DEG_RPT, 128), jnp.float32),
            pltpu.VMEM((1, 128), jnp.int32),
            pltpu.VMEM_SHARED((NPAD, 128), jnp.float32),
        ],
        compiler_params=pltpu.CompilerParams(needs_layout_passes=False),
    )


def _agg_body(m_ref, bsrc_ref, bdst_ref, cnt_ref, zeros_ref, out_ref,
              sbuf, vbuf, rows, cbuf, acc, gsem, ssem):
    c = lax.axis_index("c")
    s = lax.axis_index("s")

    cN = c * N

    for p in range(NPASS):
        pltpu.sync_copy(bsrc_ref.at[p, s], sbuf)
        pltpu.sync_copy(bdst_ref.at[p, s], vbuf)
        pltpu.sync_copy(cnt_ref.at[p, s], cbuf)

        cnt = cbuf[pl.ds(0, L)][0]
        ng = jnp.maximum(lax.shift_right_logical(cnt + (GSZ - 1), 9), 1)

        @pl.loop(0, BCH)
        def _(i):
            for v in range(K // L):
                sl = pl.ds(v * L, L)
                sbuf[i, sl] = sbuf[i, sl] + cN

        for b in range(NBUF):
            pltpu.async_copy(m_ref.at[sbuf.at[b]], rows.at[b], gsem.at[b])

        pltpu.sync_copy(zeros_ref, acc.at[pl.ds(s * ROWS_PT, ROWS_PT)])

        plsc.subcore_barrier()

        @pl.loop(0, ng)
        def _(g):
            for b in range(NBUF):
                i = g * NBUF + b
                pltpu.make_async_copy(
                    m_ref.at[sbuf.at[i]], rows.at[b], gsem.at[b]).wait()
                pltpu.async_copy(rows.at[b], acc.at[vbuf.at[i]], ssem.at[b],
                                 add=True)
            for b in range(NBUF):
                i = g * NBUF + b
                pltpu.make_async_copy(
                    rows.at[b], acc.at[vbuf.at[i]], ssem.at[b]).wait()

                @pl.when(g < ng - 1)
                def _():
                    inext = (g + 1) * NBUF + b
                    pltpu.async_copy(
                        m_ref.at[sbuf.at[inext]], rows.at[b], gsem.at[b])

        plsc.subcore_barrier()
        pltpu.sync_copy(acc.at[pl.ds(s * ROWS_PT, ROWS_PT)],
                        out_ref.at[c, p, pl.ds(s * ROWS_PT, ROWS_PT)])
        if p + 1 < NPASS:
            plsc.subcore_barrier()


@functools.cache
def _agg_call():
    return pl.kernel(
        _agg_body,
        out_type=jax.ShapeDtypeStruct((NC, NPASS, AROWS, DH), jnp.float32),
        mesh=_mesh(),
        scratch_types=[
            pltpu.VMEM((BCH, K), jnp.int32),
            pltpu.VMEM((BCH, K), jnp.int32),
            pltpu.VMEM((NBUF, K, DH), jnp.float32),
            pltpu.VMEM((L,), jnp.int32),
            pltpu.VMEM_SHARED((AROWS, DH), jnp.float32),
            pltpu.SemaphoreType.DMA((NBUF,)),
            pltpu.SemaphoreType.DMA((NBUF,)),
        ],
        compiler_params=pltpu.CompilerParams(needs_layout_passes=False),
    )


def _norm(dcol):
    return jnp.where(dcol > 0, lax.rsqrt(dcol), 0.0)


def _mm1_body(x_ref, w_ref, nod_ref, o_ref):
    no = _norm(nod_ref[0])
    hw = jnp.dot(x_ref[...], w_ref[...], preferred_element_type=jnp.float32)
    o_ref[...] = hw * no


def _mm1(features, w1, outdeg_col):
    return pl.pallas_call(
        _mm1_body,
        grid=(NB, NC),
        in_specs=[
            pl.BlockSpec((RB, D), lambda i, c: (i, 0)),
            pl.BlockSpec((D, DH), lambda i, c: (0, c)),
            pl.BlockSpec((1, RB, 1), lambda i, c: (i, 0, 0)),
        ],
        out_specs=pl.BlockSpec((RB, DH), lambda i, c: (c * NB + i, 0)),
        out_shape=jax.ShapeDtypeStruct((2 * N, DH), jnp.float32),
    )(features, w1, outdeg_col)


def _half_spec(ci):
    return pl.BlockSpec((1, 1, RB, DH),
                        lambda *g: (ci, g[0] // BPP, g[0] % BPP, 0))


def _mm2_body(a0_ref, a1_ref, nid_ref, b1_ref, w_ref, nod_ref, o_ref):
    ni = _norm(nid_ref[0])
    no = _norm(nod_ref[0])
    h = jnp.concatenate([a0_ref[0, 0], a1_ref[0, 0]], axis=1)
    h = jnp.maximum(h * ni + b1_ref[...], 0.0)
    hw = jnp.dot(h, w_ref[...], preferred_element_type=jnp.float32)
    o_ref[...] = hw * no


def _mm2(agg1, indeg_col, b1_2d, w2, outdeg_col):
    return pl.pallas_call(
        _mm2_body,
        grid=(NB, NC),
        in_specs=[
            _half_spec(0),
            _half_spec(1),
            pl.BlockSpec((1, RB, 1), lambda i, c: (i, 0, 0)),
            pl.BlockSpec((1, D), lambda i, c: (0, 0)),
            pl.BlockSpec((D, DH), lambda i, c: (0, c)),
            pl.BlockSpec((1, RB, 1), lambda i, c: (i, 0, 0)),
        ],
        out_specs=pl.BlockSpec((RB, DH), lambda i, c: (c * NB + i, 0)),
        out_shape=jax.ShapeDtypeStruct((2 * N, DH), jnp.float32),
    )(agg1, agg1, indeg_col, b1_2d, w2, outdeg_col)


def _epi_body(a0_ref, a1_ref, nid_ref, b2_ref, o_ref):
    ni = _norm(nid_ref[0])
    agg = jnp.concatenate([a0_ref[0, 0], a1_ref[0, 0]], axis=1)
    o_ref[...] = agg * ni + b2_ref[...]


def _epi(agg2, indeg_col, b2_2d):
    return pl.pallas_call(
        _epi_body,
        grid=(NB,),
        in_specs=[
            _half_spec(0),
            _half_spec(1),
            pl.BlockSpec((1, RB, 1), lambda i: (i, 0, 0)),
            pl.BlockSpec((1, D), lambda i: (0, 0)),
        ],
        out_specs=pl.BlockSpec((RB, D), lambda i: (i, 0)),
        out_shape=jax.ShapeDtypeStruct((N, D), jnp.float32),
    )(agg2, agg2, indeg_col, b2_2d)


def kernel(features, edge_index, W1, b1, W2, b2):
    ei_flat = edge_index.reshape(-1)

    bsrc, bdst, bcnt, deg = _bin_call()(ei_flat)
    outdeg_col = deg[0].reshape(-1)[:N].reshape(NB, RB, 1)
    indeg_col = deg[1].reshape(-1)[:N].reshape(NB, RB, 1)
    bsrc = bsrc.reshape(NPASS, NS, BCH, K)
    bdst = bdst.reshape(NPASS, NS, BCH, K)

    zeros = jnp.zeros((ROWS_PT, DH), jnp.float32)
    m1 = _mm1(features, W1, outdeg_col)
    agg1 = _agg_call()(m1, bsrc, bdst, bcnt, zeros)
    m2 = _mm2(agg1, indeg_col, b1.reshape(1, D), W2, outdeg_col)
    agg2 = _agg_call()(m2, bsrc, bdst, bcnt, zeros)
    return _epi(agg2, indeg_col, b2.reshape(1, D))
